# baseline (device time: 232710 ns/iter reference)
import jax
import jax.numpy as jnp
from jax import lax
from jax.experimental import pallas as pl
from jax.experimental.pallas import tpu as pltpu

B, S, H, Dh, Dr = 4, 256, 32, 128, 64
D = 4096
DC_HALF = 128
BS = B * S
SCALE = (Dh + Dr) ** -0.5

F32 = jnp.float32
BF16 = jnp.bfloat16


def _c_exchange_q(x2d, Wdkv, Wuk, Wuv, Wq):
    nb = 8
    blk = D // nb

    def body(x_ref, wdkv_ref, wuk_ref, wuv_ref, wq_ref,
             xbf_ref, cc_ref, wukf_ref, wuvf_ref, q_ref,
             send_sems, recv_sems):
        j = pl.program_id(0)
        my_x = lax.axis_index("x")
        my_y = lax.axis_index("y")
        my_z = lax.axis_index("z")
        peer = (my_x, 1 - my_y, my_z)

        def make_rdmas(lo, hi):
            srcs = [cc_ref.at[:, lo:hi], wukf_ref.at[lo:hi, :],
                    wuvf_ref.at[lo:hi, :]]
            return [
                pltpu.make_async_remote_copy(
                    src_ref=src, dst_ref=src,
                    send_sem=send_sems.at[i], recv_sem=recv_sems.at[i],
                    device_id=peer, device_id_type=pl.DeviceIdType.MESH,
                )
                for i, src in enumerate(srcs)
            ]

        def per_half(fn):
            @pl.when(my_y == 0)
            def _():
                fn(0, DC_HALF)

            @pl.when(my_y == 1)
            def _():
                fn(DC_HALF, 2 * DC_HALF)

        @pl.when(j == 0)
        def _():
            barrier = pltpu.get_barrier_semaphore()
            pl.semaphore_signal(barrier, inc=1, device_id=peer,
                                device_id_type=pl.DeviceIdType.MESH)
            pl.semaphore_wait(barrier, 1)

            xbf_ref[...] = x_ref[...].astype(BF16)
            my_c = jnp.dot(
                xbf_ref[...], wdkv_ref[...].astype(BF16),
                preferred_element_type=F32,
            ).astype(BF16)

            def fill_and_send(lo, hi):
                cc_ref[:, lo:hi] = my_c
                wukf_ref[lo:hi, :] = wuk_ref[...].astype(BF16)
                wuvf_ref[lo:hi, :] = wuv_ref[...].astype(BF16)
                for rdma in make_rdmas(lo, hi):
                    rdma.start()

            per_half(fill_and_send)

        q_ref[...] = (jnp.dot(xbf_ref[...], wq_ref[...].astype(BF16),
                              preferred_element_type=F32)
                      * SCALE).astype(BF16)

        @pl.when(j == nb - 1)
        def _():
            def wait_all(lo, hi):
                for rdma in make_rdmas(lo, hi):
                    rdma.wait()

            per_half(wait_all)

    return pl.pallas_call(
        body,
        grid=(nb,),
        in_specs=[
            pl.BlockSpec((BS, D), lambda j: (0, 0)),
            pl.BlockSpec((D, DC_HALF), lambda j: (0, 0)),
            pl.BlockSpec((DC_HALF, D), lambda j: (0, 0)),
            pl.BlockSpec((DC_HALF, D), lambda j: (0, 0)),
            pl.BlockSpec((D, blk), lambda j: (0, j)),
        ],
        out_specs=[
            pl.BlockSpec((BS, D), lambda j: (0, 0)),
            pl.BlockSpec((BS, 2 * DC_HALF), lambda j: (0, 0)),
            pl.BlockSpec((2 * DC_HALF, D), lambda j: (0, 0)),
            pl.BlockSpec((2 * DC_HALF, D), lambda j: (0, 0)),
            pl.BlockSpec((BS, blk), lambda j: (0, j)),
        ],
        out_shape=[
            jax.ShapeDtypeStruct((BS, D), BF16),
            jax.ShapeDtypeStruct((BS, 2 * DC_HALF), BF16),
            jax.ShapeDtypeStruct((2 * DC_HALF, D), BF16),
            jax.ShapeDtypeStruct((2 * DC_HALF, D), BF16),
            jax.ShapeDtypeStruct((BS, D), BF16),
        ],
        scratch_shapes=[
            pltpu.SemaphoreType.DMA((3,)),
            pltpu.SemaphoreType.DMA((3,)),
        ],
        compiler_params=pltpu.CompilerParams(
            collective_id=0, vmem_limit_bytes=100 * 1024 * 1024,
        ),
    )(x2d, Wdkv, Wuk, Wuv, Wq)


def _kv(cc, Wukf, Wuvf):
    nb = 8
    blk = D // nb
    dc = 2 * DC_HALF

    def body(cc_ref, wuk_ref, wuv_ref, k_ref, v_ref):
        c_ = cc_ref[...]
        k_ref[...] = jnp.dot(c_, wuk_ref[...],
                             preferred_element_type=F32).astype(BF16)
        v_ref[...] = jnp.dot(c_, wuv_ref[...],
                             preferred_element_type=F32).astype(BF16)

    return pl.pallas_call(
        body,
        grid=(nb,),
        in_specs=[
            pl.BlockSpec((BS, dc), lambda j: (0, 0)),
            pl.BlockSpec((dc, blk), lambda j: (0, j)),
            pl.BlockSpec((dc, blk), lambda j: (0, j)),
        ],
        out_specs=[
            pl.BlockSpec((BS, blk), lambda j: (0, j)),
            pl.BlockSpec((BS, blk), lambda j: (0, j)),
        ],
        out_shape=[
            jax.ShapeDtypeStruct((BS, D), BF16),
            jax.ShapeDtypeStruct((BS, D), BF16),
        ],
    )(cc, Wukf, Wuvf)


def _matmul(a_bf, w, n_blk, out_dtype, scale=None):
    m, k = a_bf.shape
    _, n = w.shape
    nb = n // n_blk

    def body(a_ref, w_ref, o_ref):
        r = jnp.dot(a_ref[...], w_ref[...].astype(BF16),
                    preferred_element_type=F32)
        if scale is not None:
            r = r * scale
        o_ref[...] = r.astype(out_dtype)

    return pl.pallas_call(
        body,
        grid=(nb,),
        in_specs=[
            pl.BlockSpec((m, k), lambda j: (0, 0)),
            pl.BlockSpec((k, n_blk), lambda j: (0, j)),
        ],
        out_specs=pl.BlockSpec((m, n_blk), lambda j: (0, j)),
        out_shape=jax.ShapeDtypeStruct((m, n), out_dtype),
    )(a_bf, w)


def _kr_proj(xbf, Wkr):

    def body(x_ref, w_ref, o_ref):
        o_ref[...] = jnp.dot(x_ref[...], w_ref[...].astype(BF16),
                             preferred_element_type=F32).astype(BF16)

    return pl.pallas_call(
        body,
        in_specs=[pl.BlockSpec(memory_space=pltpu.VMEM)] * 2,
        out_specs=pl.BlockSpec(memory_space=pltpu.VMEM),
        out_shape=jax.ShapeDtypeStruct((BS, Dr), BF16),
    )(xbf, Wkr)


def _attention_out(Q2, K2, V2, Qr2, Kr, Wo):
    nb = 16
    blk = D // nb

    def body(q_ref, k_ref, v_ref, qr_ref, kr_ref, wo_ref, out_ref, o_ref):
        j = pl.program_id(0)

        @pl.when(j == 0)
        def _():
            dn = (((1,), (1,)), ((), ()))
            for b in range(B):
                r = slice(b * S, (b + 1) * S)
                kr = kr_ref[r, :]
                for h in range(H):
                    ch = slice(h * Dh, (h + 1) * Dh)
                    cr = slice(h * Dr, (h + 1) * Dr)
                    q = q_ref[r, ch]
                    k = k_ref[r, ch]
                    v = v_ref[r, ch]
                    qr = qr_ref[r, cr]
                    s = (lax.dot_general(q, k, dn, preferred_element_type=F32)
                         + lax.dot_general(qr, kr, dn,
                                           preferred_element_type=F32))
                    p = jnp.exp(s)
                    p = p / jnp.sum(p, axis=-1, keepdims=True)
                    o_ref[r, ch] = jnp.dot(p.astype(BF16), v,
                                           preferred_element_type=F32
                                           ).astype(BF16)

        out_ref[...] = jnp.dot(o_ref[...], wo_ref[...].astype(BF16),
                               preferred_element_type=F32)

    return pl.pallas_call(
        body,
        grid=(nb,),
        in_specs=[
            pl.BlockSpec((BS, D), lambda j: (0, 0)),
            pl.BlockSpec((BS, D), lambda j: (0, 0)),
            pl.BlockSpec((BS, D), lambda j: (0, 0)),
            pl.BlockSpec((BS, H * Dr), lambda j: (0, 0)),
            pl.BlockSpec((BS, Dr), lambda j: (0, 0)),
            pl.BlockSpec((D, blk), lambda j: (0, j)),
        ],
        out_specs=pl.BlockSpec((BS, blk), lambda j: (0, j)),
        out_shape=jax.ShapeDtypeStruct((BS, D), F32),
        scratch_shapes=[pltpu.VMEM((BS, D), BF16)],
        compiler_params=pltpu.CompilerParams(
            vmem_limit_bytes=100 * 1024 * 1024,
        ),
    )(Q2, K2, V2, Qr2, Kr, Wo)


def kernel(x, Wdkv, Wuk, Wuv, Wq, Wqr, Wkr, Wo):
    x2d = x.reshape(BS, D)

    xbf, cc, Wukf, Wuvf, Q = _c_exchange_q(x2d, Wdkv, Wuk, Wuv, Wq)
    K, V = _kv(cc, Wukf, Wuvf)

    Qr = _matmul(xbf, Wqr, 512, BF16, SCALE)
    Kr = _kr_proj(xbf, Wkr)

    out = _attention_out(Q, K, V, Qr, Kr, Wo)
    return out.reshape(B, S, D)


# device time: 194093 ns/iter; 1.1990x vs baseline; 1.1990x over previous
import jax
import jax.numpy as jnp
from jax import lax
from jax.experimental import pallas as pl
from jax.experimental.pallas import tpu as pltpu

B, S, H, Dh, Dr = 4, 256, 32, 128, 64
D = 4096
DC_HALF = 128
BS = B * S
SCALE = (Dh + Dr) ** -0.5

F32 = jnp.float32
BF16 = jnp.bfloat16


def _c_exchange_q(x2d, Wdkv, Wuk, Wuv, Wq):
    nb = 8
    blk = D // nb

    def body(x_ref, wdkv_ref, wuk_ref, wuv_ref, wq_ref,
             xbf_ref, cc_ref, wukf_ref, wuvf_ref, q_ref,
             send_sems, recv_sems):
        j = pl.program_id(0)
        my_x = lax.axis_index("x")
        my_y = lax.axis_index("y")
        my_z = lax.axis_index("z")
        peer = (my_x, 1 - my_y, my_z)

        def make_rdmas(lo, hi):
            srcs = [cc_ref.at[:, lo:hi], wukf_ref.at[lo:hi, :],
                    wuvf_ref.at[lo:hi, :]]
            return [
                pltpu.make_async_remote_copy(
                    src_ref=src, dst_ref=src,
                    send_sem=send_sems.at[i], recv_sem=recv_sems.at[i],
                    device_id=peer, device_id_type=pl.DeviceIdType.MESH,
                )
                for i, src in enumerate(srcs)
            ]

        def per_half(fn):
            @pl.when(my_y == 0)
            def _():
                fn(0, DC_HALF)

            @pl.when(my_y == 1)
            def _():
                fn(DC_HALF, 2 * DC_HALF)

        @pl.when(j == 0)
        def _():
            barrier = pltpu.get_barrier_semaphore()
            pl.semaphore_signal(barrier, inc=1, device_id=peer,
                                device_id_type=pl.DeviceIdType.MESH)
            pl.semaphore_wait(barrier, 1)

            xbf_ref[...] = x_ref[...].astype(BF16)
            my_c = jnp.dot(
                xbf_ref[...], wdkv_ref[...].astype(BF16),
                preferred_element_type=F32,
            ).astype(BF16)

            def fill_and_send(lo, hi):
                cc_ref[:, lo:hi] = my_c
                wukf_ref[lo:hi, :] = wuk_ref[...].astype(BF16)
                wuvf_ref[lo:hi, :] = wuv_ref[...].astype(BF16)
                for rdma in make_rdmas(lo, hi):
                    rdma.start()

            per_half(fill_and_send)

        q_ref[...] = (jnp.dot(xbf_ref[...], wq_ref[...].astype(BF16),
                              preferred_element_type=F32)
                      * SCALE).astype(BF16)

        @pl.when(j == nb - 1)
        def _():
            def wait_all(lo, hi):
                for rdma in make_rdmas(lo, hi):
                    rdma.wait()

            per_half(wait_all)

    return pl.pallas_call(
        body,
        grid=(nb,),
        in_specs=[
            pl.BlockSpec((BS, D), lambda j: (0, 0)),
            pl.BlockSpec((D, DC_HALF), lambda j: (0, 0)),
            pl.BlockSpec((DC_HALF, D), lambda j: (0, 0)),
            pl.BlockSpec((DC_HALF, D), lambda j: (0, 0)),
            pl.BlockSpec((D, blk), lambda j: (0, j)),
        ],
        out_specs=[
            pl.BlockSpec((BS, D), lambda j: (0, 0)),
            pl.BlockSpec((BS, 2 * DC_HALF), lambda j: (0, 0)),
            pl.BlockSpec((2 * DC_HALF, D), lambda j: (0, 0)),
            pl.BlockSpec((2 * DC_HALF, D), lambda j: (0, 0)),
            pl.BlockSpec((BS, blk), lambda j: (0, j)),
        ],
        out_shape=[
            jax.ShapeDtypeStruct((BS, D), BF16),
            jax.ShapeDtypeStruct((BS, 2 * DC_HALF), BF16),
            jax.ShapeDtypeStruct((2 * DC_HALF, D), BF16),
            jax.ShapeDtypeStruct((2 * DC_HALF, D), BF16),
            jax.ShapeDtypeStruct((BS, D), BF16),
        ],
        scratch_shapes=[
            pltpu.SemaphoreType.DMA((3,)),
            pltpu.SemaphoreType.DMA((3,)),
        ],
        compiler_params=pltpu.CompilerParams(
            collective_id=0, vmem_limit_bytes=100 * 1024 * 1024,
        ),
    )(x2d, Wdkv, Wuk, Wuv, Wq)


def _kv_attention(cc, Wukf, Wuvf, Q2, Qr2, Kr):
    nb = H // 2
    blk = 2 * Dh
    dc = 2 * DC_HALF

    def body(cc_ref, wuk_ref, wuv_ref, q_ref, qr_ref, kr_ref, o_ref):
        c_ = cc_ref[...]
        k2 = jnp.dot(c_, wuk_ref[...],
                     preferred_element_type=F32).astype(BF16)
        v2 = jnp.dot(c_, wuv_ref[...],
                     preferred_element_type=F32).astype(BF16)
        dn = (((1,), (1,)), ((), ()))
        for b in range(B):
            r = slice(b * S, (b + 1) * S)
            kr = kr_ref[r, :]
            for hh in range(2):
                ch = slice(hh * Dh, (hh + 1) * Dh)
                cr = slice(hh * Dr, (hh + 1) * Dr)
                q = q_ref[r, ch]
                k = k2[r, ch]
                v = v2[r, ch]
                qr = qr_ref[r, cr]
                s = (lax.dot_general(q, k, dn, preferred_element_type=F32)
                     + lax.dot_general(qr, kr, dn, preferred_element_type=F32))
                p = jnp.exp(s)
                p = p / jnp.sum(p, axis=-1, keepdims=True)
                o_ref[r, ch] = jnp.dot(p.astype(BF16), v,
                                       preferred_element_type=F32).astype(BF16)

    return pl.pallas_call(
        body,
        grid=(nb,),
        in_specs=[
            pl.BlockSpec((BS, dc), lambda j: (0, 0)),
            pl.BlockSpec((dc, blk), lambda j: (0, j)),
            pl.BlockSpec((dc, blk), lambda j: (0, j)),
            pl.BlockSpec((BS, blk), lambda j: (0, j)),
            pl.BlockSpec((BS, 2 * Dr), lambda j: (0, j)),
            pl.BlockSpec((BS, Dr), lambda j: (0, 0)),
        ],
        out_specs=pl.BlockSpec((BS, blk), lambda j: (0, j)),
        out_shape=jax.ShapeDtypeStruct((BS, D), BF16),
    )(cc, Wukf, Wuvf, Q2, Qr2, Kr)


def _matmul(a_bf, w, n_blk, out_dtype, scale=None):
    m, k = a_bf.shape
    _, n = w.shape
    nb = n // n_blk

    def body(a_ref, w_ref, o_ref):
        r = jnp.dot(a_ref[...], w_ref[...].astype(BF16),
                    preferred_element_type=F32)
        if scale is not None:
            r = r * scale
        o_ref[...] = r.astype(out_dtype)

    return pl.pallas_call(
        body,
        grid=(nb,),
        in_specs=[
            pl.BlockSpec((m, k), lambda j: (0, 0)),
            pl.BlockSpec((k, n_blk), lambda j: (0, j)),
        ],
        out_specs=pl.BlockSpec((m, n_blk), lambda j: (0, j)),
        out_shape=jax.ShapeDtypeStruct((m, n), out_dtype),
    )(a_bf, w)


def _qr_kr(xbf, Wqr, Wkr):
    nb = 4
    blk = (H * Dr) // nb

    def body(x_ref, wqr_ref, wkr_ref, qr_ref, kr_ref):
        @pl.when(pl.program_id(0) == 0)
        def _():
            kr_ref[...] = jnp.dot(x_ref[...], wkr_ref[...].astype(BF16),
                                  preferred_element_type=F32).astype(BF16)

        qr_ref[...] = (jnp.dot(x_ref[...], wqr_ref[...].astype(BF16),
                               preferred_element_type=F32)
                       * SCALE).astype(BF16)

    return pl.pallas_call(
        body,
        grid=(nb,),
        in_specs=[
            pl.BlockSpec((BS, D), lambda j: (0, 0)),
            pl.BlockSpec((D, blk), lambda j: (0, j)),
            pl.BlockSpec((D, Dr), lambda j: (0, 0)),
        ],
        out_specs=[
            pl.BlockSpec((BS, blk), lambda j: (0, j)),
            pl.BlockSpec((BS, Dr), lambda j: (0, 0)),
        ],
        out_shape=[
            jax.ShapeDtypeStruct((BS, H * Dr), BF16),
            jax.ShapeDtypeStruct((BS, Dr), BF16),
        ],
    )(xbf, Wqr, Wkr)


def kernel(x, Wdkv, Wuk, Wuv, Wq, Wqr, Wkr, Wo):
    x2d = x.reshape(BS, D)

    xbf, cc, Wukf, Wuvf, Q = _c_exchange_q(x2d, Wdkv, Wuk, Wuv, Wq)
    Qr, Kr = _qr_kr(xbf, Wqr, Wkr)

    O = _kv_attention(cc, Wukf, Wuvf, Q, Qr, Kr)

    out = _matmul(O, Wo, 512, F32)
    return out.reshape(B, S, D)
